# baseline (speedup 1.0000x reference)
import functools

import jax
import jax.numpy as jnp
from jax import lax
from jax.experimental import pallas as pl
from jax.experimental.pallas import tpu as pltpu
from jax.experimental.pallas import tpu_sc as plsc

NC = 2
NS = 16
NW = NC * NS
L = 16


def _transpose_kernel(vocab, dim):
    n_blocks = vocab // 128
    tail = vocab % 128
    n_rows_out = vocab * dim // 128
    per_w, extra = divmod(n_blocks, NW)

    mesh = plsc.VectorSubcoreMesh(core_axis_name="c", subcore_axis_name="s")

    @functools.partial(
        pl.kernel,
        mesh=mesh,
        out_type=jax.ShapeDtypeStruct((n_rows_out, 128), jnp.float32),
        compiler_params=pltpu.CompilerParams(use_tc_tiling_on_sc=True, needs_layout_passes=False),
        scratch_types=[
            pltpu.VMEM((dim, 128), jnp.float32),
            pltpu.VMEM((dim, 128), jnp.float32),
        ],
    )
    def tr(table_t, tail_t, ta_out, in_blk, out_blk):
        wid = lax.axis_index("s") * NC + lax.axis_index("c")
        lo = wid * per_w + jnp.minimum(wid, extra)
        cnt = per_w + (wid < extra).astype(jnp.int32)

        row_iota = lax.iota(jnp.int32, L)

        def xpose(n_a):
            for a in range(n_a):
                for l in range(128 // L):
                    r_idx = row_iota + (l % 2) * L
                    c_idx = jnp.full((L,), 4 * a + l // 2, jnp.int32)
                    val = plsc.load_gather(in_blk, [r_idx, c_idx])
                    out_blk[a, l * L:(l + 1) * L] = val

        def body(k, carry):
            col0 = pl.multiple_of(k * 128, 128)
            pltpu.sync_copy(table_t.at[:, pl.ds(col0, 128)], in_blk)
            xpose(dim)
            pltpu.sync_copy(
                out_blk,
                ta_out.at[pl.ds(pl.multiple_of(col0 // 4, 8), dim), :])
            return carry

        lax.fori_loop(lo, lo + cnt, body, 0)

        if tail:
            @pl.when(wid == NW - 1)
            def _():
                pltpu.sync_copy(tail_t, in_blk)
                xpose(dim)
                pltpu.sync_copy(
                    out_blk,
                    ta_out.at[pl.ds((vocab - 128) // 4, dim), :])

    return tr


def _gather_kernel(total, n_rows_ta, dim):
    b_per_w = total // NW
    CH = 256
    n_chunks = b_per_w // CH
    n_groups = CH // L

    mesh = plsc.VectorSubcoreMesh(core_axis_name="c", subcore_axis_name="s")

    @functools.partial(
        pl.kernel,
        mesh=mesh,
        out_type=jax.ShapeDtypeStruct((dim, total), jnp.float32),
        compiler_params=pltpu.CompilerParams(use_tc_tiling_on_sc=True, needs_layout_passes=False),
        scratch_types=[
            pltpu.VMEM((b_per_w,), jnp.int32),
            pltpu.VMEM((CH,), jnp.int32),
            pltpu.VMEM((CH,), jnp.int32),
            pltpu.VMEM((CH, 128), jnp.float32),
            pltpu.VMEM((dim, CH), jnp.float32),
            pltpu.SemaphoreType.DMA,
        ],
    )
    def ga(idx_hbm, ta_hbm, out_hbm, idx_v, g_v, s32_v, staging, out_blk, sem):
        wid = lax.axis_index("s") * NC + lax.axis_index("c")
        base = wid * b_per_w
        pltpu.sync_copy(idx_hbm.at[pl.ds(base, b_per_w)], idx_v)

        row_iota = lax.iota(jnp.int32, L)

        def body(ch, carry):
            for m in range(n_groups):
                iv = idx_v[pl.ds(ch * CH + m * L, L)]
                g_v[m * L:(m + 1) * L] = lax.shift_right_logical(iv, 2)
                s32_v[m * L:(m + 1) * L] = lax.shift_left(
                    jnp.bitwise_and(iv, 3), 5)
            cp0 = pltpu.async_copy(
                ta_hbm.at[g_v.at[pl.ds(0, 128)]],
                staging.at[pl.ds(0, 128), :], sem)
            cp1 = pltpu.async_copy(
                ta_hbm.at[g_v.at[pl.ds(128, 128)]],
                staging.at[pl.ds(128, 128), :], sem)
            cp0.wait()
            cp1.wait()
            for m in range(n_groups):
                r_idx = row_iota + m * L
                s32 = s32_v[pl.ds(m * L, L)]
                for c in range(dim):
                    val = plsc.load_gather(staging, [r_idx, s32 + c])
                    out_blk[c, m * L:(m + 1) * L] = val
            pltpu.sync_copy(
                out_blk,
                out_hbm.at[:, pl.ds(pl.multiple_of(base + ch * CH, 128), CH)])
            return carry

        lax.fori_loop(0, n_chunks, body, 0)

    return ga


def kernel(indices, row_splits, table):
    total = indices.shape[0]
    vocab, dim = table.shape
    n_rows_ta = vocab * dim // 128
    ta = _transpose_kernel(vocab, dim)(table.T, table[vocab - 128:].T)
    out_t = _gather_kernel(total, n_rows_ta, dim)(indices, ta)
    return out_t.T


# speedup vs baseline: 1.3818x; 1.3818x over previous
import functools

import jax
import jax.numpy as jnp
from jax import lax
from jax.experimental import pallas as pl
from jax.experimental.pallas import tpu as pltpu
from jax.experimental.pallas import tpu_sc as plsc

NC = 2
NS = 16
NW = NC * NS
L = 16
BC = 256


def _transpose_kernel(vocab, dim):
    n_blocks = vocab // BC
    tail = vocab % BC
    n_rows_out = vocab * dim // 128
    n_pairs = n_blocks // 2
    per_w, extra = divmod(n_pairs, NW)
    assert n_blocks % 2 == 0

    mesh = plsc.VectorSubcoreMesh(core_axis_name="c", subcore_axis_name="s")

    @functools.partial(
        pl.kernel,
        mesh=mesh,
        out_type=jax.ShapeDtypeStruct((n_rows_out, 128), jnp.float32),
        compiler_params=pltpu.CompilerParams(
            use_tc_tiling_on_sc=True, needs_layout_passes=False),
        scratch_types=[
            pltpu.VMEM((dim, BC), jnp.float32),
            pltpu.VMEM((dim, BC), jnp.float32),
            pltpu.VMEM((BC // 4, 128), jnp.float32),
            pltpu.VMEM((BC // 4, 128), jnp.float32),
            pltpu.SemaphoreType.DMA,
            pltpu.SemaphoreType.DMA,
            pltpu.SemaphoreType.DMA,
            pltpu.SemaphoreType.DMA,
        ],
    )
    def tr(table_t, tail_t, ta_out, in0, in1, out0, out1,
           isem0, isem1, osem0, osem1):
        wid = lax.axis_index("s") * NC + lax.axis_index("c")
        lo = (wid * per_w + jnp.minimum(wid, extra)) * 2
        cnt = per_w + (wid < extra).astype(jnp.int32)

        row_iota = lax.iota(jnp.int32, L)
        ins = (in0, in1)
        outs = (out0, out1)
        isems = (isem0, isem1)
        osems = (osem0, osem1)

        def fire_in(b, p):
            col0 = pl.multiple_of(b * BC, 128)
            pltpu.async_copy(table_t.at[:, pl.ds(col0, BC)], ins[p], isems[p])

        def wait_in(p):
            pltpu.make_async_copy(
                table_t.at[:, pl.ds(0, BC)], ins[p], isems[p]).wait()

        def fire_out(b, p):
            g0 = pl.multiple_of(b * (BC // 4), 8)
            pltpu.async_copy(
                outs[p], ta_out.at[pl.ds(g0, BC // 4), :], osems[p])

        def wait_out(p):
            pltpu.make_async_copy(
                outs[p], ta_out.at[pl.ds(0, BC // 4), :], osems[p]).wait()

        zeros = jnp.zeros((L,), jnp.int32)

        def xpose(p, n_a):
            def xbody(a, carry):
                a_row = zeros + a
                a4 = 4 * a
                for l in range(128 // L):
                    r_idx = row_iota + (l % 2) * L
                    c_idx = zeros + (a4 + l // 2)
                    val = plsc.load_gather(ins[p], [r_idx, c_idx])
                    plsc.store_scatter(
                        outs[p], [a_row, row_iota + l * L], val)
                return carry

            lax.fori_loop(0, n_a, xbody, 0)

        fire_in(lo, 0)
        fire_in(lo + 1, 1)

        def body(t, carry):
            for p in range(2):
                b = lo + 2 * t + p
                wait_in(p)

                @pl.when(t > 0)
                def _():
                    wait_out(p)

                xpose(p, BC // 4)
                fire_out(b, p)

                @pl.when(t + 1 < cnt)
                def _():
                    fire_in(b + 2, p)
            return carry

        lax.fori_loop(0, cnt, body, 0)
        wait_out(0)
        wait_out(1)

        if tail:
            @pl.when(wid == NW - 1)
            def _():
                pltpu.sync_copy(tail_t, in0.at[:, pl.ds(0, 128)])
                xpose(0, 32)
                pltpu.sync_copy(
                    out0.at[pl.ds(0, 32), :],
                    ta_out.at[pl.ds((vocab - 128) // 4, 32), :])

    return tr


def _gather_kernel(total, dim):
    b_per_w = total // NW
    CH = 256
    n_chunks = b_per_w // CH
    n_groups = CH // L

    mesh = plsc.VectorSubcoreMesh(core_axis_name="c", subcore_axis_name="s")

    @functools.partial(
        pl.kernel,
        mesh=mesh,
        out_type=jax.ShapeDtypeStruct((dim, total), jnp.float32),
        compiler_params=pltpu.CompilerParams(
            use_tc_tiling_on_sc=True, needs_layout_passes=False),
        scratch_types=[
            pltpu.VMEM((b_per_w,), jnp.int32),
            pltpu.VMEM((CH,), jnp.int32),
            pltpu.VMEM((CH,), jnp.int32),
            pltpu.VMEM((CH,), jnp.int32),
            pltpu.VMEM((CH,), jnp.int32),
            pltpu.VMEM((CH, 128), jnp.float32),
            pltpu.VMEM((CH, 128), jnp.float32),
            pltpu.VMEM((dim, CH), jnp.float32),
            pltpu.VMEM((dim, CH), jnp.float32),
            pltpu.SemaphoreType.DMA,
            pltpu.SemaphoreType.DMA,
            pltpu.SemaphoreType.DMA,
            pltpu.SemaphoreType.DMA,
        ],
    )
    def ga(idx_hbm, ta_hbm, out_hbm, idx_v, g0, g1, s0, s1,
           st0, st1, ob0, ob1, gsem0, gsem1, osem0, osem1):
        wid = lax.axis_index("s") * NC + lax.axis_index("c")
        base = wid * b_per_w
        pltpu.sync_copy(idx_hbm.at[pl.ds(base, b_per_w)], idx_v)

        row_iota = lax.iota(jnp.int32, L)
        gs = (g0, g1)
        ss = (s0, s1)
        stags = (st0, st1)
        obs = (ob0, ob1)
        gsems = (gsem0, gsem1)
        osems = (osem0, osem1)

        def prep_and_fire(ch, p):
            for m in range(n_groups):
                iv = idx_v[pl.ds(ch * CH + m * L, L)]
                gs[p][m * L:(m + 1) * L] = lax.shift_right_logical(iv, 2)
                ss[p][m * L:(m + 1) * L] = lax.shift_left(
                    jnp.bitwise_and(iv, 3), 5)
            pltpu.async_copy(ta_hbm.at[gs[p].at[pl.ds(0, 128)]],
                             stags[p].at[pl.ds(0, 128), :], gsems[p])
            pltpu.async_copy(ta_hbm.at[gs[p].at[pl.ds(128, 128)]],
                             stags[p].at[pl.ds(128, 128), :], gsems[p])

        def wait_gather(p):
            pltpu.make_async_copy(
                ta_hbm.at[pl.ds(0, 128), :],
                stags[p].at[pl.ds(0, 128), :], gsems[p]).wait()
            pltpu.make_async_copy(
                ta_hbm.at[pl.ds(0, 128), :],
                stags[p].at[pl.ds(128, 128), :], gsems[p]).wait()

        def wait_out(p):
            pltpu.make_async_copy(
                obs[p], out_hbm.at[:, pl.ds(0, CH)], osems[p]).wait()

        def work(ch, p):
            @pl.when(ch + 1 < n_chunks)
            def _():
                prep_and_fire(ch + 1, 1 - p)

            wait_gather(p)

            @pl.when(ch >= 2)
            def _():
                wait_out(p)

            def ebody(m, carry):
                r_idx = row_iota + m * L
                s32 = ss[p][pl.ds(m * L, L)]
                col = row_iota + m * L
                for c in range(dim):
                    val = plsc.load_gather(stags[p], [r_idx, s32 + c])
                    plsc.store_scatter(
                        obs[p], [jnp.full((L,), c, jnp.int32), col], val)
                return carry

            lax.fori_loop(0, n_groups, ebody, 0)
            pltpu.async_copy(
                obs[p],
                out_hbm.at[:, pl.ds(pl.multiple_of(base + ch * CH, 128), CH)],
                osems[p])

        prep_and_fire(0, 0)

        def body(ch, carry):
            @pl.when(ch % 2 == 0)
            def _():
                work(ch, 0)

            @pl.when(ch % 2 == 1)
            def _():
                work(ch, 1)
            return carry

        lax.fori_loop(0, n_chunks, body, 0)
        wait_out(0)
        wait_out(1)

    return ga


def kernel(indices, row_splits, table):
    total = indices.shape[0]
    vocab, dim = table.shape
    ta = _transpose_kernel(vocab, dim)(table.T, table[vocab - 128:].T)
    out_t = _gather_kernel(total, dim)(indices, ta)
    return out_t.T


# speedup vs baseline: 1.9555x; 1.4152x over previous
import functools

import jax
import jax.numpy as jnp
from jax import lax
from jax.experimental import pallas as pl
from jax.experimental.pallas import tpu as pltpu
from jax.experimental.pallas import tpu_sc as plsc

NC = 2
NS = 16
NW = NC * NS
L = 16
BC = 256


def _transpose_kernel(vocab, dim):
    n_blocks = vocab // BC
    tail = vocab % BC
    n_rows_out = vocab * dim // 128
    n_pairs = n_blocks // 2
    per_w, extra = divmod(n_pairs, NW)
    assert n_blocks % 2 == 0

    mesh = plsc.VectorSubcoreMesh(core_axis_name="c", subcore_axis_name="s")

    @functools.partial(
        pl.kernel,
        mesh=mesh,
        out_type=jax.ShapeDtypeStruct((n_rows_out, 128), jnp.float32),
        compiler_params=pltpu.CompilerParams(
            use_tc_tiling_on_sc=True, needs_layout_passes=False),
        scratch_types=[
            pltpu.VMEM((dim, BC), jnp.float32),
            pltpu.VMEM((dim, BC), jnp.float32),
            pltpu.VMEM((BC // 4, 128), jnp.float32),
            pltpu.VMEM((BC // 4, 128), jnp.float32),
            pltpu.SemaphoreType.DMA,
            pltpu.SemaphoreType.DMA,
            pltpu.SemaphoreType.DMA,
            pltpu.SemaphoreType.DMA,
        ],
    )
    def tr(table_t, tail_t, ta_out, in0, in1, out0, out1,
           isem0, isem1, osem0, osem1):
        wid = lax.axis_index("s") * NC + lax.axis_index("c")
        lo = (wid * per_w + jnp.minimum(wid, extra)) * 2
        cnt = per_w + (wid < extra).astype(jnp.int32)

        row_iota = lax.iota(jnp.int32, L)
        ins = (in0, in1)
        outs = (out0, out1)
        isems = (isem0, isem1)
        osems = (osem0, osem1)

        def fire_in(b, p):
            col0 = pl.multiple_of(b * BC, 128)
            pltpu.async_copy(table_t.at[:, pl.ds(col0, BC)], ins[p], isems[p])

        def wait_in(p):
            pltpu.make_async_copy(
                table_t.at[:, pl.ds(0, BC)], ins[p], isems[p]).wait()

        def fire_out(b, p):
            g0 = pl.multiple_of(b * (BC // 4), 8)
            pltpu.async_copy(
                outs[p], ta_out.at[pl.ds(g0, BC // 4), :], osems[p])

        def wait_out(p):
            pltpu.make_async_copy(
                outs[p], ta_out.at[pl.ds(0, BC // 4), :], osems[p]).wait()

        zeros = jnp.zeros((L,), jnp.int32)

        def xpose(p, n_a):
            def xbody(a, carry):
                a_row = zeros + a
                a4 = 4 * a
                for l in range(128 // L):
                    r_idx = row_iota + (l % 2) * L
                    c_idx = zeros + (a4 + l // 2)
                    val = plsc.load_gather(ins[p], [r_idx, c_idx])
                    plsc.store_scatter(
                        outs[p], [a_row, row_iota + l * L], val)
                return carry

            lax.fori_loop(0, n_a, xbody, 0)

        fire_in(lo, 0)
        fire_in(lo + 1, 1)

        def body(t, carry):
            for p in range(2):
                b = lo + 2 * t + p
                wait_in(p)

                @pl.when(t > 0)
                def _():
                    wait_out(p)

                xpose(p, BC // 4)
                fire_out(b, p)

                @pl.when(t + 1 < cnt)
                def _():
                    fire_in(b + 2, p)
            return carry

        lax.fori_loop(0, cnt, body, 0)
        wait_out(0)
        wait_out(1)

        if tail:
            @pl.when(wid == NW - 1)
            def _():
                pltpu.sync_copy(tail_t, in0.at[:, pl.ds(0, 128)])
                xpose(0, 32)
                pltpu.sync_copy(
                    out0.at[pl.ds(0, 32), :],
                    ta_out.at[pl.ds((vocab - 128) // 4, 32), :])

    return tr


def _gather_kernel(total, dim):
    b_per_w = total // NW
    CH = 256
    n_chunks = b_per_w // CH
    n_groups = CH // L

    mesh = plsc.VectorSubcoreMesh(core_axis_name="c", subcore_axis_name="s")

    @functools.partial(
        pl.kernel,
        mesh=mesh,
        out_type=jax.ShapeDtypeStruct((dim, total), jnp.float32),
        compiler_params=pltpu.CompilerParams(
            use_tc_tiling_on_sc=True, needs_layout_passes=False),
        scratch_types=[
            pltpu.VMEM((b_per_w,), jnp.int32),
            pltpu.VMEM((CH,), jnp.int32),
            pltpu.VMEM((CH,), jnp.int32),
            pltpu.VMEM((CH,), jnp.int32),
            pltpu.VMEM((CH,), jnp.int32),
            pltpu.VMEM((CH, 128), jnp.float32),
            pltpu.VMEM((CH, 128), jnp.float32),
            pltpu.VMEM((dim, CH), jnp.float32),
            pltpu.VMEM((dim, CH), jnp.float32),
            pltpu.SemaphoreType.DMA,
            pltpu.SemaphoreType.DMA,
            pltpu.SemaphoreType.DMA,
            pltpu.SemaphoreType.DMA,
        ],
    )
    def ga(idx_hbm, ta_hbm, out_hbm, idx_v, g0, g1, s0, s1,
           st0, st1, ob0, ob1, gsem0, gsem1, osem0, osem1):
        wid = lax.axis_index("s") * NC + lax.axis_index("c")
        base = wid * b_per_w
        pltpu.sync_copy(idx_hbm.at[pl.ds(base, b_per_w)], idx_v)

        row_iota = lax.iota(jnp.int32, L)
        gs = (g0, g1)
        ss = (s0, s1)
        stags = (st0, st1)
        obs = (ob0, ob1)
        gsems = (gsem0, gsem1)
        osems = (osem0, osem1)

        def prep_and_fire(ch, p):
            for m in range(n_groups):
                iv = idx_v[pl.ds(ch * CH + m * L, L)]
                gs[p][m * L:(m + 1) * L] = lax.shift_right_logical(iv, 2)
                ss[p][m * L:(m + 1) * L] = lax.shift_left(
                    jnp.bitwise_and(iv, 3), 5)
            pltpu.async_copy(ta_hbm.at[gs[p].at[pl.ds(0, 128)]],
                             stags[p].at[pl.ds(0, 128), :], gsems[p])
            pltpu.async_copy(ta_hbm.at[gs[p].at[pl.ds(128, 128)]],
                             stags[p].at[pl.ds(128, 128), :], gsems[p])

        def wait_gather(p):
            pltpu.make_async_copy(
                ta_hbm.at[pl.ds(0, 128), :],
                stags[p].at[pl.ds(0, 128), :], gsems[p]).wait()
            pltpu.make_async_copy(
                ta_hbm.at[pl.ds(0, 128), :],
                stags[p].at[pl.ds(128, 128), :], gsems[p]).wait()

        def wait_out(p):
            pltpu.make_async_copy(
                obs[p], out_hbm.at[:, pl.ds(0, CH)], osems[p]).wait()

        def work(ch, p):
            @pl.when(ch + 1 < n_chunks)
            def _():
                prep_and_fire(ch + 1, 1 - p)

            wait_gather(p)

            @pl.when(ch >= 2)
            def _():
                wait_out(p)

            def ebody(m, carry):
                r_idx = row_iota + m * L
                s32 = ss[p][pl.ds(m * L, L)]
                col = row_iota + m * L
                for c in range(dim):
                    val = plsc.load_gather(stags[p], [r_idx, s32 + c])
                    plsc.store_scatter(
                        obs[p], [jnp.full((L,), c, jnp.int32), col], val)
                return carry

            lax.fori_loop(0, n_groups, ebody, 0)
            pltpu.async_copy(
                obs[p],
                out_hbm.at[:, pl.ds(pl.multiple_of(base + ch * CH, 128), CH)],
                osems[p])

        prep_and_fire(0, 0)

        def body(ch, carry):
            @pl.when(ch % 2 == 0)
            def _():
                work(ch, 0)

            @pl.when(ch % 2 == 1)
            def _():
                work(ch, 1)
            return carry

        lax.fori_loop(0, n_chunks, body, 0)
        wait_out(0)
        wait_out(1)

    return ga


def kernel(indices, row_splits, table):
    total = indices.shape[0]
    vocab, dim = table.shape
    ta = table.reshape(vocab * dim // 128, 128)
    out_t = _gather_kernel(total, dim)(indices, ta)
    return out_t.T


# speedup vs baseline: 2.0742x; 1.0607x over previous
import functools

import jax
import jax.numpy as jnp
from jax import lax
from jax.experimental import pallas as pl
from jax.experimental.pallas import tpu as pltpu
from jax.experimental.pallas import tpu_sc as plsc

DIM = 32
NC = 2
NS = 16
NW = NC * NS
K = 128
CH_ROWS = 10
C = CH_ROWS * K


def _make_gather(total, vocab, dim):
    b_per_w = total // NW
    n_idx_rows = b_per_w // K
    n_chunks = n_idx_rows // CH_ROWS

    mesh = plsc.VectorSubcoreMesh(core_axis_name="c", subcore_axis_name="s")

    @functools.partial(
        pl.kernel,
        mesh=mesh,
        out_type=jax.ShapeDtypeStruct((total, dim), jnp.float32),
        compiler_params=pltpu.CompilerParams(use_tc_tiling_on_sc=False),
        scratch_types=[
            pltpu.VMEM((b_per_w,), jnp.int32),
            pltpu.VMEM((C, dim), jnp.float32),
            pltpu.VMEM((C, dim), jnp.float32),
            pltpu.SemaphoreType.DMA,
            pltpu.SemaphoreType.DMA,
            pltpu.SemaphoreType.DMA,
            pltpu.SemaphoreType.DMA,
        ],
    )
    def gather_kernel(idx_hbm, table_hbm, out_hbm, idx_v, buf0, buf1,
                      gsem0, gsem1, osem0, osem1):
        wid = lax.axis_index("s") * NC + lax.axis_index("c")
        base = wid * b_per_w
        pltpu.sync_copy(idx_hbm.at[pl.ds(base, b_per_w)], idx_v)

        bufs = (buf0, buf1)
        gsems = (gsem0, gsem1)
        osems = (osem0, osem1)

        gathers = [None] * n_chunks
        writes = [None] * n_chunks

        def fire(c):
            buf, sem = bufs[c % 2], gsems[c % 2]
            cps = []
            for j in range(CH_ROWS):
                cps.append(
                    pltpu.async_copy(
                        table_hbm.at[idx_v.at[pl.ds((c * CH_ROWS + j) * K, K)]],
                        buf.at[pl.ds(j * K, K)],
                        sem,
                    )
                )
            gathers[c] = cps

        fire(0)
        for c in range(n_chunks):
            if c + 1 < n_chunks:
                nb = (c + 1) % 2
                if writes[nb] is not None:
                    writes[nb].wait()
                    writes[nb] = None
                fire(c + 1)
            for cp in gathers[c]:
                cp.wait()
            writes[c % 2] = pltpu.async_copy(
                bufs[c % 2],
                out_hbm.at[pl.ds(base + c * C, C)],
                osems[c % 2],
            )
        for w in writes:
            if w is not None:
                w.wait()

    return gather_kernel


def kernel(indices, row_splits, table):
    total = indices.shape[0]
    vocab, dim = table.shape
    return _make_gather(total, vocab, dim)(indices, table)
